# baseline (speedup 1.0000x reference)
import functools

import jax
import jax.numpy as jnp
from jax import lax
from jax.experimental import pallas as pl
from jax.experimental.pallas import tpu as pltpu
from jax.experimental.pallas import tpu_sc as plsc

_NW = 32
_C = 1024
_NBUF = 2


@functools.lru_cache(maxsize=None)
def _build_gather(n_s, n_b, dim):
    n = n_s * n_b
    n_per_w = n // _NW
    q_per_w = n_per_w // _C
    cps = n_b // _C
    mesh = plsc.VectorSubcoreMesh(core_axis_name="c", subcore_axis_name="s")

    @functools.partial(
        pl.kernel,
        mesh=mesh,
        out_type=jax.ShapeDtypeStruct((n_s, n_b, dim), jnp.float32),
        scratch_types=[
            pltpu.VMEM((n_per_w,), jnp.int32),
            pltpu.VMEM((_NBUF, _C, dim), jnp.float32),
            pltpu.SemaphoreType.DMA((_NBUF,)),
            pltpu.SemaphoreType.DMA((_NBUF,)),
        ],
        compiler_params=pltpu.CompilerParams(use_tc_tiling_on_sc=False),
    )
    def gather_kernel(idx_hbm, table_hbm, out_hbm, idx_v, rows_v, gsem, osem):
        wid = lax.axis_index("s") * 2 + lax.axis_index("c")
        base = wid * n_per_w
        q0 = wid * q_per_w
        pltpu.sync_copy(idx_hbm.at[pl.ds(base, n_per_w)], idx_v)

        def g_desc(g, b):
            return pltpu.make_async_copy(
                table_hbm.at[idx_v.at[pl.ds(g * _C, _C)]],
                rows_v.at[b],
                gsem.at[b],
            )

        def o_desc(g, b):
            q = q0 + g
            return pltpu.make_async_copy(
                rows_v.at[b],
                out_hbm.at[q // cps, pl.ds((q % cps) * _C, _C)],
                osem.at[b],
            )

        for b in range(_NBUF):
            g_desc(b, b).start()

        def body(t, carry):
            for b in range(_NBUF):
                g = t * _NBUF + b
                g_desc(g, b).wait()
                o_desc(g, b).start()
                o_desc(g, b).wait()

                @pl.when(g + _NBUF < q_per_w)
                def _():
                    g_desc(g + _NBUF, b).start()

            return carry

        lax.fori_loop(0, (q_per_w + _NBUF - 1) // _NBUF - 1, body, 0)

        for g in range(_NBUF * ((q_per_w + _NBUF - 1) // _NBUF - 1), q_per_w):
            b = g % _NBUF
            g_desc(g, b).wait()
            o_desc(g, b).start()
            o_desc(g, b).wait()

    return gather_kernel


def kernel(x, planes_weight):
    b, s = x.shape
    _, dim = planes_weight.shape
    idx = x.T.reshape(b * s).astype(jnp.int32)
    out3 = _build_gather(s, b, dim)(idx, planes_weight)
    return out3.transpose(1, 0, 2)
